# baseline (speedup 1.0000x reference)
import functools

import jax
import jax.numpy as jnp
from jax.experimental import pallas as pl
from jax.experimental.pallas import tpu as pltpu

EPS = 1e-5
F32 = jnp.float32
BF16 = jnp.bfloat16


def _ln_rows(xr, w, b):
    m = jnp.mean(xr, axis=-1, keepdims=True)
    xc = xr - m
    v = jnp.mean(xc * xc, axis=-1, keepdims=True)
    return xc * jax.lax.rsqrt(v + EPS) * w + b


def _attn_fused_body(h, dh, ncs, x_ref, w_ref, lw_ref, lb_ref, e_ref,
                     et_ref, g_ref, s0_ref, gf_ref, y_ref, fs_ref,
                     xn_ref, qkv_ref, st_ref):
    m = pl.program_id(0)
    nbm = pl.num_programs(0)

    @pl.when(m == 0)
    def _():
        st_ref[...] = s0_ref[...]

    bt = x_ref.shape[0]
    d = lw_ref.shape[1]
    xn_ref[...] = _ln_rows(x_ref[...], lw_ref[...], lb_ref[...]).astype(BF16)
    xn = xn_ref[...]

    cs = bt // ncs
    row = jax.lax.broadcasted_iota(jnp.int32, (cs, cs), 0)
    col = jax.lax.broadcasted_iota(jnp.int32, (cs, cs), 1)
    causal = row >= col
    riota = jax.lax.broadcasted_iota(jnp.int32, (cs, 1), 0)

    hg = 512 // dh
    for g in range(d // 512):
        c0 = g * 512
        qkv_ref[:, 3 * c0:3 * c0 + 512] = jnp.dot(
            xn, w_ref[:, c0:c0 + 512],
            preferred_element_type=F32).astype(BF16)
        qkv_ref[:, 3 * c0 + 512:3 * c0 + 1024] = jnp.dot(
            xn, w_ref[:, d + c0:d + c0 + 512],
            preferred_element_type=F32).astype(BF16)
        qkv_ref[:, 3 * c0 + 1024:3 * c0 + 1536] = jnp.dot(
            xn, w_ref[:, 2 * d + c0:2 * d + c0 + 512],
            preferred_element_type=F32).astype(BF16)
        for sub in range(ncs):
            base = sub * cs
            tmul = ((m * ncs + sub) * cs + 1 + riota).astype(F32)
            qf = qkv_ref[base:base + cs, 3 * c0:3 * c0 + 512]
            kf = qkv_ref[base:base + cs, 3 * c0 + 512:3 * c0 + 1024]
            vf = qkv_ref[base:base + cs, 3 * c0 + 1024:3 * c0 + 1536]
            qs = jnp.dot(qf, e_ref[c0:c0 + 512, :],
                         preferred_element_type=F32)
            og = (tmul * jnp.dot(qs.astype(BF16), et_ref[:, c0:c0 + 512],
                                 preferred_element_type=F32)
                  ) * g_ref[:, c0:c0 + 512]
            outs = []
            for j8 in range(hg):
                j = g * hg + j8
                qj = qf[:, j8 * dh:(j8 + 1) * dh]
                kj = kf[:, j8 * dh:(j8 + 1) * dh]
                vj = vf[:, j8 * dh:(j8 + 1) * dh]
                s = jax.lax.dot_general(qj, vj, (((1,), (1,)), ((), ())),
                                        preferred_element_type=F32)
                sm = jnp.where(causal, s, 0.0).astype(BF16)
                intra = jnp.dot(sm, kj, preferred_element_type=F32)
                inter = jax.lax.dot_general(qj, st_ref[j].astype(BF16),
                                            (((1,), (1,)), ((), ())),
                                            preferred_element_type=F32)
                outs.append(intra + inter)
                st_ref[j] = st_ref[j] + jax.lax.dot_general(
                    kj, vj, (((0,), (0,)), ((), ())),
                    preferred_element_type=F32)
            y_ref[base:base + cs, c0:c0 + 512] = (
                x_ref[base:base + cs, c0:c0 + 512] + og
                + jnp.concatenate(outs, axis=1))

    @pl.when(m == nbm - 1)
    def _():
        fs_ref[...] = st_ref[...] + gf_ref[...]


def _mlp_body(d, y_ref, w_ref, wd_ref, lw_ref, lb_ref, o_ref, x2_ref):
    x2_ref[...] = _ln_rows(y_ref[...], lw_ref[...], lb_ref[...]).astype(BF16)
    x2 = x2_ref[...]
    acc = y_ref[...]
    for nb in range(0, d, 512):
        gate = jnp.dot(x2, w_ref[:, nb:nb + 512],
                       preferred_element_type=F32)
        up = jnp.dot(x2, w_ref[:, d + nb:d + nb + 512],
                     preferred_element_type=F32)
        a_slice = (up / ((1.0 + jnp.exp(-up)) * (1.0 + jnp.exp(-gate)))
                   ).astype(BF16)
        acc = acc + jnp.dot(a_slice, wd_ref[nb:nb + 512, :],
                            preferred_element_type=F32)
    o_ref[...] = acc


def kernel(x, state, Wq, Wk, Wv, gamma, Wgate, Wdown, ln1_w, ln1_b,
           ln2_w, ln2_b):
    t, d = x.shape
    h, dh, _ = state.shape

    bt = min(512, t)
    cs = min(256, t)
    nbm, nc = t // bt, t // cs

    ln1w = ln1_w.reshape(1, d)
    ln1b = ln1_b.reshape(1, d)
    ln2w = ln2_w.reshape(1, d)
    ln2b = ln2_b.reshape(1, d)
    grow = gamma.reshape(1, d)
    lane = jnp.arange(d, dtype=jnp.int32)
    emat = (lane[:, None] // dh == jnp.arange(h, dtype=jnp.int32)[None, :]
            ).astype(BF16)
    etmat = (jnp.arange(h, dtype=jnp.int32)[:, None] == lane[None, :] // dh
             ).astype(BF16)
    gfin = jnp.broadcast_to((t * gamma)[:, :, None], (h, dh, dh))
    wqkv16 = jnp.concatenate(
        [Wq.astype(BF16), Wk.astype(BF16), Wv.astype(BF16)], axis=1)
    wg16, wd16 = Wgate.astype(BF16), Wdown.astype(BF16)

    cp = pltpu.CompilerParams(
        dimension_semantics=("arbitrary",),
        vmem_limit_bytes=100 * 1024 * 1024,
    )

    ncs = bt // cs
    y, fs = pl.pallas_call(
        functools.partial(_attn_fused_body, h, dh, ncs),
        grid=(nbm,),
        in_specs=[
            pl.BlockSpec((bt, d), lambda m: (m, 0)),
            pl.BlockSpec((d, 3 * d), lambda m: (0, 0)),
            pl.BlockSpec((1, d), lambda m: (0, 0)),
            pl.BlockSpec((1, d), lambda m: (0, 0)),
            pl.BlockSpec((d, h), lambda m: (0, 0)),
            pl.BlockSpec((h, d), lambda m: (0, 0)),
            pl.BlockSpec((1, d), lambda m: (0, 0)),
            pl.BlockSpec((h, dh, dh), lambda m: (0, 0, 0)),
            pl.BlockSpec((h, dh, dh), lambda m: (0, 0, 0)),
        ],
        out_specs=[
            pl.BlockSpec((bt, d), lambda m: (m, 0)),
            pl.BlockSpec((h, dh, dh), lambda m: (0, 0, 0)),
        ],
        out_shape=[
            jax.ShapeDtypeStruct((t, d), F32),
            jax.ShapeDtypeStruct((h, dh, dh), F32),
        ],
        scratch_shapes=[
            pltpu.VMEM((bt, d), BF16),
            pltpu.VMEM((bt, 3 * d), BF16),
            pltpu.VMEM((h, dh, dh), F32),
        ],
        compiler_params=cp,
    )(x, wqkv16, ln1w, ln1b, emat, etmat, grow, state, gfin)

    out = pl.pallas_call(
        functools.partial(_mlp_body, d),
        grid=(nbm,),
        in_specs=[
            pl.BlockSpec((bt, d), lambda m: (m, 0)),
            pl.BlockSpec((d, 2 * d), lambda m: (0, 0)),
            pl.BlockSpec((d, d), lambda m: (0, 0)),
            pl.BlockSpec((1, d), lambda m: (0, 0)),
            pl.BlockSpec((1, d), lambda m: (0, 0)),
        ],
        out_specs=pl.BlockSpec((bt, d), lambda m: (m, 0)),
        out_shape=jax.ShapeDtypeStruct((t, d), F32),
        scratch_shapes=[pltpu.VMEM((bt, d), BF16)],
        compiler_params=cp,
    )(y, wg16, wd16, ln2w, ln2b)

    return out, fs


# speedup vs baseline: 1.0230x; 1.0230x over previous
import functools

import jax
import jax.numpy as jnp
from jax.experimental import pallas as pl
from jax.experimental.pallas import tpu as pltpu

EPS = 1e-5
F32 = jnp.float32
BF16 = jnp.bfloat16


def _ln_rows(xr, w, b):
    m = jnp.mean(xr, axis=-1, keepdims=True)
    xc = xr - m
    v = jnp.mean(xc * xc, axis=-1, keepdims=True)
    return xc * jax.lax.rsqrt(v + EPS) * w + b


def _attn_fused_body(h, dh, ncs, x_ref, w_ref, lw_ref, lb_ref, e_ref,
                     et_ref, g_ref, s0_ref, gf_ref, y_ref, fs_ref,
                     xn_ref, qkv_ref, st_ref):
    m = pl.program_id(0)
    nbm = pl.num_programs(0)

    @pl.when(m == 0)
    def _():
        st_ref[...] = s0_ref[...]

    bt = x_ref.shape[0]
    d = lw_ref.shape[1]
    xn_ref[...] = _ln_rows(x_ref[...], lw_ref[...], lb_ref[...]).astype(BF16)
    xn = xn_ref[...]
    for nb in range(0, 3 * d, 512):
        qkv_ref[:, nb:nb + 512] = jnp.dot(
            xn, w_ref[:, nb:nb + 512],
            preferred_element_type=F32).astype(BF16)

    cs = bt // ncs
    row = jax.lax.broadcasted_iota(jnp.int32, (cs, cs), 0)
    col = jax.lax.broadcasted_iota(jnp.int32, (cs, cs), 1)
    causal = row >= col
    riota = jax.lax.broadcasted_iota(jnp.int32, (cs, 1), 0)

    for sub in range(ncs):
        base = sub * cs
        tmul = ((m * ncs + sub) * cs + 1 + riota).astype(F32)
        qf = qkv_ref[base:base + cs, 0:d]
        kf = qkv_ref[base:base + cs, d:2 * d]
        vf = qkv_ref[base:base + cs, 2 * d:3 * d]
        qs_all = jnp.dot(qf, e_ref[...], preferred_element_type=F32)
        og_full = (tmul * jnp.dot(qs_all.astype(BF16), et_ref[...],
                                  preferred_element_type=F32)) * g_ref[...]
        outs = []
        for j in range(h):
            qj = qf[:, j * dh:(j + 1) * dh]
            kj = kf[:, j * dh:(j + 1) * dh]
            vj = vf[:, j * dh:(j + 1) * dh]
            s = jax.lax.dot_general(qj, vj, (((1,), (1,)), ((), ())),
                                    preferred_element_type=F32)
            sm = jnp.where(causal, s, 0.0).astype(BF16)
            intra = jnp.dot(sm, kj, preferred_element_type=F32)
            inter = jax.lax.dot_general(qj, st_ref[j].astype(BF16),
                                        (((1,), (1,)), ((), ())),
                                        preferred_element_type=F32)
            outs.append(intra + inter)
            st_ref[j] = st_ref[j] + jax.lax.dot_general(
                kj, vj, (((0,), (0,)), ((), ())),
                preferred_element_type=F32)
        y_ref[base:base + cs, :] = (x_ref[base:base + cs, :] + og_full
                                    + jnp.concatenate(outs, axis=1))

    @pl.when(m == nbm - 1)
    def _():
        fs_ref[...] = st_ref[...] + gf_ref[...]


def _mlp_body(d, y_ref, w_ref, wd_ref, lw_ref, lb_ref, o_ref, x2_ref):
    x2_ref[...] = _ln_rows(y_ref[...], lw_ref[...], lb_ref[...]).astype(BF16)
    x2 = x2_ref[...]
    acc = y_ref[...]
    for nb in range(0, d, 512):
        gate = jnp.dot(x2, w_ref[:, nb:nb + 512],
                       preferred_element_type=F32)
        up = jnp.dot(x2, w_ref[:, d + nb:d + nb + 512],
                     preferred_element_type=F32)
        a_slice = (up / ((1.0 + jnp.exp(-up)) * (1.0 + jnp.exp(-gate)))
                   ).astype(BF16)
        acc = acc + jnp.dot(a_slice, wd_ref[nb:nb + 512, :],
                            preferred_element_type=F32)
    o_ref[...] = acc


def kernel(x, state, Wq, Wk, Wv, gamma, Wgate, Wdown, ln1_w, ln1_b,
           ln2_w, ln2_b):
    t, d = x.shape
    h, dh, _ = state.shape

    bt = min(512, t)
    cs = min(256, t)
    nbm, nc = t // bt, t // cs

    ln1w = ln1_w.reshape(1, d)
    ln1b = ln1_b.reshape(1, d)
    ln2w = ln2_w.reshape(1, d)
    ln2b = ln2_b.reshape(1, d)
    grow = gamma.reshape(1, d)
    lane = jnp.arange(d, dtype=jnp.int32)
    emat = (lane[:, None] // dh == jnp.arange(h, dtype=jnp.int32)[None, :]
            ).astype(BF16)
    etmat = (jnp.arange(h, dtype=jnp.int32)[:, None] == lane[None, :] // dh
             ).astype(BF16)
    gfin = jnp.broadcast_to((t * gamma)[:, :, None], (h, dh, dh))
    wqkv16 = jnp.concatenate(
        [Wq.astype(BF16), Wk.astype(BF16), Wv.astype(BF16)], axis=1)
    wg16, wd16 = Wgate.astype(BF16), Wdown.astype(BF16)

    cp = pltpu.CompilerParams(
        dimension_semantics=("arbitrary",),
        vmem_limit_bytes=100 * 1024 * 1024,
    )

    ncs = bt // cs
    y, fs = pl.pallas_call(
        functools.partial(_attn_fused_body, h, dh, ncs),
        grid=(nbm,),
        in_specs=[
            pl.BlockSpec((bt, d), lambda m: (m, 0)),
            pl.BlockSpec((d, 3 * d), lambda m: (0, 0)),
            pl.BlockSpec((1, d), lambda m: (0, 0)),
            pl.BlockSpec((1, d), lambda m: (0, 0)),
            pl.BlockSpec((d, h), lambda m: (0, 0)),
            pl.BlockSpec((h, d), lambda m: (0, 0)),
            pl.BlockSpec((1, d), lambda m: (0, 0)),
            pl.BlockSpec((h, dh, dh), lambda m: (0, 0, 0)),
            pl.BlockSpec((h, dh, dh), lambda m: (0, 0, 0)),
        ],
        out_specs=[
            pl.BlockSpec((bt, d), lambda m: (m, 0)),
            pl.BlockSpec((h, dh, dh), lambda m: (0, 0, 0)),
        ],
        out_shape=[
            jax.ShapeDtypeStruct((t, d), F32),
            jax.ShapeDtypeStruct((h, dh, dh), F32),
        ],
        scratch_shapes=[
            pltpu.VMEM((bt, d), BF16),
            pltpu.VMEM((bt, 3 * d), BF16),
            pltpu.VMEM((h, dh, dh), F32),
        ],
        compiler_params=cp,
    )(x, wqkv16, ln1w, ln1b, emat, etmat, grow, state, gfin)

    out = pl.pallas_call(
        functools.partial(_mlp_body, d),
        grid=(nbm,),
        in_specs=[
            pl.BlockSpec((bt, d), lambda m: (m, 0)),
            pl.BlockSpec((d, 2 * d), lambda m: (0, 0)),
            pl.BlockSpec((d, d), lambda m: (0, 0)),
            pl.BlockSpec((1, d), lambda m: (0, 0)),
            pl.BlockSpec((1, d), lambda m: (0, 0)),
        ],
        out_specs=pl.BlockSpec((bt, d), lambda m: (m, 0)),
        out_shape=jax.ShapeDtypeStruct((t, d), F32),
        scratch_shapes=[pltpu.VMEM((bt, d), BF16)],
        compiler_params=cp,
    )(y, wg16, wd16, ln2w, ln2b)

    return out, fs
